# baseline (speedup 1.0000x reference)
import functools

import jax
import jax.numpy as jnp
from jax import lax
from jax.experimental import pallas as pl
from jax.experimental.pallas import tpu as pltpu
from jax.experimental.pallas import tpu_sc as plsc

_MAXLAG = 30
_NLAGS = 5
_PADT = 32
_NB1 = 16


def _acf_part_kernel(x_ref, out_ref, xct_ref, yt_ref, *, t, bb):
    xt = jnp.transpose(x_ref[...])
    mu = jnp.mean(xt, axis=0, keepdims=True)
    xct = xt - mu
    var = jnp.sum(xct * xct, axis=0, keepdims=True)
    yt = xct / (var + 1e-8)
    xct_ref[:t, :] = xct
    yt_ref[:t, :] = yt
    yt_ref[t:, :] = jnp.zeros((_PADT, bb), jnp.float32)
    lane = lax.broadcasted_iota(jnp.int32, (1, 128), 1)
    acc = jnp.zeros((1, 128), jnp.float32)
    for lag in range(1, _MAXLAG + 1):
        prod = yt_ref[pl.ds(lag, t), :] * xct_ref[:t, :]
        r = jnp.sum(prod.reshape(t // 8, 8, bb), axis=0)
        contrib = jnp.sum(r)
        acc = acc + jnp.where(lane == lag - 1, contrib, 0.0)
    out_ref[...] = acc.reshape(1, 1, 128)


def _sc_topk_kernel(parts_hbm, lags_hbm, parts_v, lags_v):
    cid = lax.axis_index("c")
    sid = lax.axis_index("s")

    @pl.when(jnp.logical_and(cid == 0, sid == 0))
    def _():
        pltpu.sync_copy(parts_hbm, parts_v)
        acc0 = jnp.zeros((16,), jnp.float32)
        acc1 = jnp.zeros((16,), jnp.float32)
        for i in range(_NB1):
            acc0 = acc0 + parts_v[i, 0:16]
            acc1 = acc1 + parts_v[i, 16:32]
        iota = lax.iota(jnp.int32, 16)
        neg = jnp.float32(-jnp.inf)
        acc1 = jnp.where(iota < (_MAXLAG - 16), acc1, neg)
        lagvec = jnp.zeros((16,), jnp.int32)
        for k in range(_NLAGS):
            m0 = jnp.max(acc0)
            m1 = jnp.max(acc1)
            use0 = m0 >= m1
            i0 = jnp.min(jnp.where(acc0 == m0, iota, 64))
            i1 = jnp.min(jnp.where(acc1 == m1, iota, 64))
            idx = jnp.where(use0, i0, i1 + 16)
            lagvec = jnp.where(iota == k, idx + 1, lagvec)
            acc0 = jnp.where(jnp.logical_and(use0, iota == i0), neg, acc0)
            acc1 = jnp.where(jnp.logical_and(jnp.logical_not(use0), iota == i1), neg, acc1)
        lags_v[...] = lagvec
        pltpu.sync_copy(lags_v, lags_hbm)


def _feat_kernel(lags_ref, x_ref, out_ref, *, bb, t):
    x = x_ref[...]
    out_ref[0] = x
    ti = lax.broadcasted_iota(jnp.int32, (bb, t), 1)
    for k in range(_NLAGS):
        lag = lags_ref[k]
        rolled = pltpu.roll(x, lag, 1)
        out_ref[k + 1] = jnp.where(ti < lag, 0.0, rolled)


def kernel(inputs):
    x = inputs
    b, t = x.shape
    bb1 = 256
    nb1 = b // bb1

    parts = pl.pallas_call(
        functools.partial(_acf_part_kernel, t=t, bb=bb1),
        grid=(nb1,),
        in_specs=[pl.BlockSpec((bb1, t), lambda i: (i, 0))],
        out_specs=pl.BlockSpec((1, 1, 128), lambda i: (i, 0, 0)),
        out_shape=jax.ShapeDtypeStruct((nb1, 1, 128), jnp.float32),
        scratch_shapes=[
            pltpu.VMEM((t + _PADT, bb1), jnp.float32),
            pltpu.VMEM((t + _PADT, bb1), jnp.float32),
        ],
        compiler_params=pltpu.CompilerParams(dimension_semantics=("parallel",)),
    )(x)

    sc_topk = functools.partial(
        pl.kernel,
        mesh=plsc.VectorSubcoreMesh(core_axis_name="c", subcore_axis_name="s"),
        out_type=jax.ShapeDtypeStruct((16,), jnp.int32),
        scratch_types=[
            pltpu.VMEM((nb1, 128), jnp.float32),
            pltpu.VMEM((16,), jnp.int32),
        ],
        compiler_params=pltpu.CompilerParams(needs_layout_passes=False),
    )(_sc_topk_kernel)
    lags = sc_topk(parts.reshape(nb1, 128))[:8]

    bb2 = 256
    nb2 = b // bb2
    planes = pl.pallas_call(
        functools.partial(_feat_kernel, bb=bb2, t=t),
        grid_spec=pltpu.PrefetchScalarGridSpec(
            num_scalar_prefetch=1,
            grid=(nb2,),
            in_specs=[pl.BlockSpec((bb2, t), lambda i, lags: (i, 0))],
            out_specs=pl.BlockSpec((_NLAGS + 1, bb2, t), lambda i, lags: (0, i, 0)),
        ),
        out_shape=jax.ShapeDtypeStruct((_NLAGS + 1, b, t), jnp.float32),
        compiler_params=pltpu.CompilerParams(dimension_semantics=("parallel",)),
    )(lags, x)

    return jnp.transpose(planes, (1, 2, 0))


# speedup vs baseline: 1.5492x; 1.5492x over previous
import functools

import jax
import jax.numpy as jnp
from jax import lax
from jax.experimental import pallas as pl
from jax.experimental.pallas import tpu as pltpu
from jax.experimental.pallas import tpu_sc as plsc

_MAXLAG = 30
_NLAGS = 5
_PADT = 32
_NB1 = 16


def _acf_part_kernel(x_ref, out_ref, xct_ref, ytr_ref, ytpad_ref, *, t, bb):
    xt = jnp.transpose(x_ref[...])
    mu = jnp.mean(xt, axis=0, keepdims=True)
    xct = xt - mu
    var = jnp.sum(xct * xct, axis=0, keepdims=True)
    yt = xct / (var + 1e-8)
    xct_ref[...] = xct
    ytpad_ref[:t, :] = yt
    ytpad_ref[t:, :] = jnp.zeros((_PADT + 8, bb), jnp.float32)
    for r in range(8):
        ytr_ref[r, :, :] = ytpad_ref[pl.ds(r, t + _PADT), :]
    accs = [jnp.zeros((1, bb), jnp.float32) for _ in range(_MAXLAG)]
    ch = 128
    for c in range(t // ch):
        xchunk = xct_ref[pl.ds(c * ch, ch), :]
        for lag in range(1, _MAXLAG + 1):
            q, r = divmod(lag, 8)
            ys = ytr_ref[r, pl.ds(8 * q + c * ch, ch), :]
            s = jnp.sum(xchunk * ys, axis=0, keepdims=True)
            accs[lag - 1] = accs[lag - 1] + s
    lane = lax.broadcasted_iota(jnp.int32, (1, 128), 1)
    acc = jnp.zeros((1, 128), jnp.float32)
    for lag in range(1, _MAXLAG + 1):
        contrib = jnp.sum(accs[lag - 1])
        acc = acc + jnp.where(lane == lag - 1, contrib, 0.0)
    out_ref[...] = acc.reshape(1, 1, 128)


def _sc_topk_kernel(parts_hbm, lags_hbm, parts_v, lags_v):
    cid = lax.axis_index("c")
    sid = lax.axis_index("s")

    @pl.when(jnp.logical_and(cid == 0, sid == 0))
    def _():
        pltpu.sync_copy(parts_hbm, parts_v)
        acc0 = jnp.zeros((16,), jnp.float32)
        acc1 = jnp.zeros((16,), jnp.float32)
        for i in range(_NB1):
            acc0 = acc0 + parts_v[i, 0:16]
            acc1 = acc1 + parts_v[i, 16:32]
        iota = lax.iota(jnp.int32, 16)
        neg = jnp.float32(-jnp.inf)
        acc1 = jnp.where(iota < (_MAXLAG - 16), acc1, neg)
        lagvec = jnp.zeros((16,), jnp.int32)
        for k in range(_NLAGS):
            m0 = jnp.max(acc0)
            m1 = jnp.max(acc1)
            use0 = m0 >= m1
            i0 = jnp.min(jnp.where(acc0 == m0, iota, 64))
            i1 = jnp.min(jnp.where(acc1 == m1, iota, 64))
            idx = jnp.where(use0, i0, i1 + 16)
            lagvec = jnp.where(iota == k, idx + 1, lagvec)
            acc0 = jnp.where(jnp.logical_and(use0, iota == i0), neg, acc0)
            acc1 = jnp.where(jnp.logical_and(jnp.logical_not(use0), iota == i1), neg, acc1)
        lags_v[...] = lagvec
        pltpu.sync_copy(lags_v, lags_hbm)


def _feat_kernel(lags_ref, x_ref, out_ref, *, bb, t):
    x = x_ref[...]
    out_ref[0] = x
    ti = lax.broadcasted_iota(jnp.int32, (bb, t), 1)
    for k in range(_NLAGS):
        lag = lags_ref[k]
        rolled = pltpu.roll(x, lag, 1)
        out_ref[k + 1] = jnp.where(ti < lag, 0.0, rolled)


def kernel(inputs):
    x = inputs
    b, t = x.shape
    bb1 = 256
    nb1 = b // bb1

    parts = pl.pallas_call(
        functools.partial(_acf_part_kernel, t=t, bb=bb1),
        grid=(nb1,),
        in_specs=[pl.BlockSpec((bb1, t), lambda i: (i, 0))],
        out_specs=pl.BlockSpec((1, 1, 128), lambda i: (i, 0, 0)),
        out_shape=jax.ShapeDtypeStruct((nb1, 1, 128), jnp.float32),
        scratch_shapes=[
            pltpu.VMEM((t, bb1), jnp.float32),
            pltpu.VMEM((8, t + _PADT, bb1), jnp.float32),
            pltpu.VMEM((t + _PADT + 8, bb1), jnp.float32),
        ],
        compiler_params=pltpu.CompilerParams(dimension_semantics=("parallel",)),
    )(x)

    sc_topk = functools.partial(
        pl.kernel,
        mesh=plsc.VectorSubcoreMesh(core_axis_name="c", subcore_axis_name="s"),
        out_type=jax.ShapeDtypeStruct((16,), jnp.int32),
        scratch_types=[
            pltpu.VMEM((nb1, 128), jnp.float32),
            pltpu.VMEM((16,), jnp.int32),
        ],
        compiler_params=pltpu.CompilerParams(needs_layout_passes=False),
    )(_sc_topk_kernel)
    lags = sc_topk(parts.reshape(nb1, 128))[:8]

    bb2 = 256
    nb2 = b // bb2
    planes = pl.pallas_call(
        functools.partial(_feat_kernel, bb=bb2, t=t),
        grid_spec=pltpu.PrefetchScalarGridSpec(
            num_scalar_prefetch=1,
            grid=(nb2,),
            in_specs=[pl.BlockSpec((bb2, t), lambda i, lags: (i, 0))],
            out_specs=pl.BlockSpec((_NLAGS + 1, bb2, t), lambda i, lags: (0, i, 0)),
        ),
        out_shape=jax.ShapeDtypeStruct((_NLAGS + 1, b, t), jnp.float32),
        compiler_params=pltpu.CompilerParams(dimension_semantics=("parallel",)),
    )(lags, x)

    return jnp.transpose(planes, (1, 2, 0))
